# baseline (speedup 1.0000x reference)
import functools

import jax
import jax.numpy as jnp
import numpy as np
from jax import lax
from jax.experimental import pallas as pl
from jax.experimental.pallas import tpu as pltpu
from jax.experimental.pallas import tpu_sc as plsc

_N = 10000
_NPAD = 10240
_H = 128
_E = 320000
_ED = 19
_B = 64
_VOCAB = 101
_VPAD = 104
_LAT = 32

_NC, _NS = 2, 16
_NW = _NC * _NS
_CHUNK = 64
_CPW = 160
_NCHUNK = _NW * _CPW
_EPAD = _NCHUNK * _CHUNK
_NPT = _NPAD // _NW

_BLK = 1024
_NB = _NPAD // _BLK
_NEB = _EPAD // _BLK
_PBLK = 128
_NPB = _NPAD // _PBLK

_sc_mesh = plsc.VectorSubcoreMesh(core_axis_name="c", subcore_axis_name="s")
_sc_params = pltpu.CompilerParams(needs_layout_passes=False)

_PERM = np.stack(
    [np.arange(16)[None, :] + np.arange(0, _H, 32)[:, None],
     np.arange(16)[None, :] + 16 + np.arange(0, _H, 32)[:, None]],
    axis=-1).reshape(-1)


def _silu(x):
    return jax.nn.silu(x)


@functools.partial(
    pl.kernel,
    out_type=jax.ShapeDtypeStruct((_NCHUNK, _CHUNK), jnp.float32),
    mesh=_sc_mesh,
    compiler_params=_sc_params,
    scratch_types=[
        pltpu.VMEM((_NPAD,), jnp.float32),
        pltpu.VMEM((_NPAD,), jnp.float32),
        pltpu.VMEM((_NPAD,), jnp.float32),
        pltpu.VMEM((1, _CHUNK), jnp.int32),
        pltpu.VMEM((1, _CHUNK), jnp.int32),
        pltpu.VMEM((1, _CHUNK), jnp.float32),
    ],
)
def _sc_dist(px_hbm, py_hbm, pz_hbm, src_hbm, dst_hbm, out_hbm,
             px_v, py_v, pz_v, sb, db, d2b):
    wid = lax.axis_index("s") * _NC + lax.axis_index("c")
    pltpu.sync_copy(px_hbm, px_v)
    pltpu.sync_copy(py_hbm, py_v)
    pltpu.sync_copy(pz_hbm, pz_v)

    def chunk(i, carry):
        ci = wid * _CPW + i
        pltpu.sync_copy(src_hbm.at[ci], sb.at[0])
        pltpu.sync_copy(dst_hbm.at[ci], db.at[0])
        for j in range(_CHUNK // 16):
            sl = pl.ds(j * 16, 16)
            si = sb[0, sl]
            di = db[0, sl]
            dx = plsc.load_gather(px_v, [si]) - plsc.load_gather(px_v, [di])
            dy = plsc.load_gather(py_v, [si]) - plsc.load_gather(py_v, [di])
            dz = plsc.load_gather(pz_v, [si]) - plsc.load_gather(pz_v, [di])
            d2b[0, sl] = dx * dx + dy * dy + dz * dz
        pltpu.sync_copy(d2b.at[0], out_hbm.at[ci])
        return carry

    lax.fori_loop(0, _CPW, chunk, 0)


@functools.partial(
    pl.kernel,
    out_type=jax.ShapeDtypeStruct((_NPAD, _H), jnp.float32),
    mesh=_sc_mesh,
    compiler_params=_sc_params,
    scratch_types=[
        pltpu.VMEM((_NPT + 8, _H), jnp.float32),
        pltpu.VMEM((1, _CHUNK), jnp.int32),
        pltpu.VMEM((1, _CHUNK), jnp.int32),
        pltpu.VMEM((1, _CHUNK), jnp.int32),
        pltpu.VMEM((1, _CHUNK), jnp.int32),
        pltpu.VMEM((_CHUNK, _H), jnp.float32),
        pltpu.VMEM((_CHUNK, _H), jnp.float32),
        pltpu.VMEM((_CHUNK, _H), jnp.float32),
        pltpu.VMEM((_CHUNK, _H), jnp.float32),
        pltpu.VMEM((48,), jnp.int32),
        pltpu.SemaphoreType.DMA,
        pltpu.SemaphoreType.DMA,
    ],
)
def _sc_edge(t_hbm, c_hbm, src_hbm, dst_hbm, bnd_hbm, out_hbm,
             accl, sv0, sv1, dv0, dv1, tv0, tv1, cv0, cv1, bndv, sem0, sem1):
    cid = lax.axis_index("c")
    sid = lax.axis_index("s")
    gtid = sid * _NC + cid
    base_node = gtid * _NPT

    pltpu.sync_copy(bnd_hbm, bndv)
    bvec = bndv[pl.ds(gtid, 16)]
    start = bvec[0]
    end = bvec[1]
    ck0 = start // _CHUNK
    nck = jnp.where(end > start, (end - 1) // _CHUNK - ck0 + 1, 0)

    def zrow(r, carry):
        for j in range(_H // 16):
            accl[r, pl.ds(j * 16, 16)] = jnp.zeros((16,), jnp.float32)
        return carry

    lax.fori_loop(0, _NPT + 8, zrow, 0)

    def stage(ck, sv, dv):
        pltpu.sync_copy(src_hbm.at[ck], sv.at[0])
        pltpu.sync_copy(dst_hbm.at[ck], dv.at[0])

    def fire(ck, sv, tv, cv, sem):
        pltpu.async_copy(t_hbm.at[sv.at[0]], tv, sem)
        pltpu.async_copy(c_hbm.at[pl.ds(ck * _CHUNK, _CHUNK)], cv, sem)

    def drain(tv, cv, sem):
        pltpu.make_async_copy(t_hbm.at[pl.ds(0, _CHUNK)], tv, sem).wait()
        pltpu.make_async_copy(c_hbm.at[pl.ds(0, _CHUNK)], cv, sem).wait()

    def process(ck, tv, cv, dv):
        gbase = ck * _CHUNK

        def grp(g, carry):
            r0 = g * 16
            lvec = dv[0, pl.ds(r0, 16)] - base_node
            for rr in range(16):
                r = r0 + rr
                pos = gbase + r
                valid = (pos >= start) & (pos < end)
                lrow = jnp.where(valid, lvec[rr], _NPT)
                for j in range(_H // 16):
                    sl = pl.ds(j * 16, 16)
                    x = tv[r, sl] + cv[r, sl]
                    accl[lrow, sl] = accl[lrow, sl] + x / (1.0 + jnp.exp(-x))
            return carry

        lax.fori_loop(0, _CHUNK // 16, grp, 0)

    @pl.when(nck > 0)
    def _prologue():
        stage(ck0, sv0, dv0)
        fire(ck0, sv0, tv0, cv0, sem0)

    def body(k, carry):
        @pl.when(k + 1 < nck)
        def _fire_next():
            @pl.when(lax.rem(k, 2) == 0)
            def _():
                stage(ck0 + k + 1, sv1, dv1)
                fire(ck0 + k + 1, sv1, tv1, cv1, sem1)

            @pl.when(lax.rem(k, 2) == 1)
            def _():
                stage(ck0 + k + 1, sv0, dv0)
                fire(ck0 + k + 1, sv0, tv0, cv0, sem0)

        @pl.when(lax.rem(k, 2) == 0)
        def _proc_even():
            drain(tv0, cv0, sem0)
            process(ck0 + k, tv0, cv0, dv0)

        @pl.when(lax.rem(k, 2) == 1)
        def _proc_odd():
            drain(tv1, cv1, sem1)
            process(ck0 + k, tv1, cv1, dv1)

        return carry

    lax.fori_loop(0, nck, body, 0)
    pltpu.sync_copy(accl.at[pl.ds(0, _NPT)], out_hbm.at[pl.ds(base_node, _NPT)])


def _full_spec(shape):
    n = len(shape)
    return pl.BlockSpec(shape, lambda *_, _n=n: (0,) * _n)


def _embed_body(z_ref, emb_ref, o_ref):
    zb = z_ref[0, 0, :]
    oh = (zb[:, None] == lax.broadcasted_iota(jnp.int32, (_BLK, _VPAD), 1)
          ).astype(jnp.float32)
    o_ref[...] = jnp.dot(oh, emb_ref[...], preferred_element_type=jnp.float32)


def _embed(z3, emb):
    return pl.pallas_call(
        _embed_body,
        grid=(_NB,),
        in_specs=[pl.BlockSpec((1, 1, _BLK), lambda i: (i, 0, 0)),
                  _full_spec((_VPAD, _H))],
        out_specs=pl.BlockSpec((_BLK, _H), lambda i: (i, 0)),
        out_shape=jax.ShapeDtypeStruct((_NPAD, _H), jnp.float32),
    )(z3, emb)


def _mm_body(x_ref, w_ref, o_ref):
    o_ref[...] = jnp.dot(x_ref[...], w_ref[...],
                         preferred_element_type=jnp.float32)


def _mm(x, w):
    k, m = w.shape
    return pl.pallas_call(
        _mm_body,
        grid=(_NB,),
        in_specs=[pl.BlockSpec((_BLK, k), lambda i: (i, 0)),
                  _full_spec((k, m))],
        out_specs=pl.BlockSpec((_BLK, m), lambda i: (i, 0)),
        out_shape=jax.ShapeDtypeStruct((_NPAD, m), jnp.float32),
    )(x, w)


def _c_body(ea_ref, d2_ref, wea_ref, wd_ref, b_ref, o_ref):
    dist = jnp.sqrt(d2_ref[...])
    o_ref[...] = (jnp.dot(ea_ref[...], wea_ref[...],
                          preferred_element_type=jnp.float32)
                  + dist * wd_ref[...] + b_ref[...])


def _edge_const(ea, d2, wea, wd, b):
    return pl.pallas_call(
        _c_body,
        grid=(_NEB,),
        in_specs=[pl.BlockSpec((_BLK, _ED), lambda i: (i, 0)),
                  pl.BlockSpec((_BLK, 1), lambda i: (i, 0)),
                  _full_spec((_ED, _H)),
                  _full_spec((1, _H)),
                  _full_spec((1, _H))],
        out_specs=pl.BlockSpec((_BLK, _H), lambda i: (i, 0)),
        out_shape=jax.ShapeDtypeStruct((_EPAD, _H), jnp.float32),
    )(ea, d2, wea, wd, b)


def _upd_body(s_ref, a_ref, w1_ref, w2_ref, b_ref, o_ref):
    s = s_ref[...]
    x = (jnp.dot(s, w1_ref[...], preferred_element_type=jnp.float32)
         + jnp.dot(a_ref[...], w2_ref[...], preferred_element_type=jnp.float32)
         + b_ref[...])
    o_ref[...] = s + _silu(x)


def _upd(s, agg, w1, w2, b):
    return pl.pallas_call(
        _upd_body,
        grid=(_NB,),
        in_specs=[pl.BlockSpec((_BLK, _H), lambda i: (i, 0)),
                  pl.BlockSpec((_BLK, _H), lambda i: (i, 0)),
                  _full_spec((_H, _H)),
                  _full_spec((_H, _H)),
                  _full_spec((1, _H))],
        out_specs=pl.BlockSpec((_BLK, _H), lambda i: (i, 0)),
        out_shape=jax.ShapeDtypeStruct((_NPAD, _H), jnp.float32),
    )(s, agg, w1, w2, b)


def _pool_body(b_ref, s_ref, sum_ref, cnt_ref, max_ref):
    i = pl.program_id(0)

    @pl.when(i == 0)
    def _init():
        sum_ref[...] = jnp.zeros_like(sum_ref)
        cnt_ref[...] = jnp.zeros_like(cnt_ref)
        max_ref[...] = jnp.full_like(max_ref, -jnp.inf)

    bb = b_ref[0, 0, :]
    s = s_ref[...]
    ohb = bb[:, None] == lax.broadcasted_iota(jnp.int32, (_PBLK, _B), 1)
    oh = ohb.astype(jnp.float32)
    sum_ref[...] += lax.dot_general(oh, s, (((0,), (0,)), ((), ())),
                                    preferred_element_type=jnp.float32)
    cnt_ref[...] += jnp.sum(oh, axis=0)[:, None]
    pen = jnp.where(ohb, 0.0, -jnp.inf)
    cand = jnp.max(pen[:, :, None] + s[:, None, :], axis=0)
    max_ref[...] = jnp.maximum(max_ref[...], cand)


def _pool(b3, s):
    out3 = jax.ShapeDtypeStruct((_B, _H), jnp.float32)
    return pl.pallas_call(
        _pool_body,
        grid=(_NPB,),
        in_specs=[pl.BlockSpec((1, 1, _PBLK), lambda i: (i, 0, 0)),
                  pl.BlockSpec((_PBLK, _H), lambda i: (i, 0))],
        out_specs=(pl.BlockSpec((_B, _H), lambda i: (0, 0)),) * 3,
        out_shape=(out3, out3, out3),
    )(b3, s)


def _poolmlp_body(sum_ref, cnt_ref, max_ref, w1a, w1b, w1c, b1, w2, b2, g_ref):
    cnt = cnt_ref[...]
    mean = sum_ref[...] / jnp.maximum(cnt, 1.0)
    mx = jnp.where(cnt > 0, max_ref[...], 0.0)
    h = (jnp.dot(mean, w1a[...], preferred_element_type=jnp.float32)
         + jnp.dot(sum_ref[...], w1b[...], preferred_element_type=jnp.float32)
         + jnp.dot(mx, w1c[...], preferred_element_type=jnp.float32)
         + b1[...])
    h = _silu(h)
    g_ref[...] = jnp.dot(h, w2[...], preferred_element_type=jnp.float32) + b2[...]


def _poolmlp(sumf, cntf, maxf, w1a, w1b, w1c, b1, w2, b2):
    fs = _full_spec
    return pl.pallas_call(
        _poolmlp_body,
        in_specs=[fs((_B, _H))] * 3 + [fs((_H, _H))] * 3 + [fs((1, _H)),
                  fs((_H, _H)), fs((1, _H))],
        out_specs=fs((_B, _H)),
        out_shape=jax.ShapeDtypeStruct((_B, _H), jnp.float32),
    )(sumf, cntf, maxf, w1a, w1b, w1c, b1, w2, b2)


def _lat_body(s_ref, b_ref, g_ref, wa, wb, b1, w2, b2, wmu, bmu, wlv, blv,
              mu_ref, lv_ref):
    s = s_ref[...]
    bb = b_ref[0, 0, :]
    oh = (bb[:, None] == lax.broadcasted_iota(jnp.int32, (_BLK, _B), 1)
          ).astype(jnp.float32)
    gb = jnp.dot(oh, g_ref[...], preferred_element_type=jnp.float32)
    h = _silu(jnp.dot(s, wa[...], preferred_element_type=jnp.float32)
              + jnp.dot(gb, wb[...], preferred_element_type=jnp.float32)
              + b1[...])
    h = _silu(jnp.dot(h, w2[...], preferred_element_type=jnp.float32) + b2[...])
    mu_ref[...] = jnp.dot(h, wmu[...], preferred_element_type=jnp.float32) + bmu[...]
    lv_ref[...] = jnp.dot(h, wlv[...], preferred_element_type=jnp.float32) + blv[...]


def _latent(s, b3, g, wa, wb, b1, w2, b2, wmu, bmu, wlv, blv):
    fs = _full_spec
    out = jax.ShapeDtypeStruct((_NPAD, _LAT), jnp.float32)
    return pl.pallas_call(
        _lat_body,
        grid=(_NB,),
        in_specs=[pl.BlockSpec((_BLK, _H), lambda i: (i, 0)),
                  pl.BlockSpec((1, 1, _BLK), lambda i: (i, 0, 0)),
                  fs((_B, _H)), fs((_H, _H)), fs((_H, _H)), fs((1, _H)),
                  fs((_H, _H)), fs((1, _H)),
                  fs((_H, _LAT)), fs((1, _LAT)), fs((_H, _LAT)), fs((1, _LAT))],
        out_specs=(pl.BlockSpec((_BLK, _LAT), lambda i: (i, 0)),) * 2,
        out_shape=(out, out),
    )(s, b3, g, wa, wb, b1, w2, b2, wmu, bmu, wlv, blv)


def _dec_body(mu_ref, w1, b1, w2, b2, w3, b3, o_ref):
    a = _silu(jnp.dot(mu_ref[...], w1[...], preferred_element_type=jnp.float32)
              + b1[...])
    a = _silu(jnp.dot(a, w2[...], preferred_element_type=jnp.float32) + b2[...])
    o_ref[...] = jnp.dot(a, w3[...], preferred_element_type=jnp.float32) + b3[...]


def _decmlp(mu, w1, b1, w2, b2, w3, b3):
    fs = _full_spec
    return pl.pallas_call(
        _dec_body,
        grid=(_NB,),
        in_specs=[pl.BlockSpec((_BLK, _LAT), lambda i: (i, 0)),
                  fs((_LAT, _H)), fs((1, _H)), fs((_H, _H)), fs((1, _H)),
                  fs((_H, _H)), fs((1, _H))],
        out_specs=pl.BlockSpec((_BLK, _H), lambda i: (i, 0)),
        out_shape=jax.ShapeDtypeStruct((_NPAD, _H), jnp.float32),
    )(mu, w1, b1, w2, b2, w3, b3)


def _coord_body(s_ref, p_ref, w1, b1, w2, b2, o_ref):
    h = _silu(jnp.dot(s_ref[...], w1[...], preferred_element_type=jnp.float32)
              + b1[...])
    delta = jnp.dot(h, w2[...], preferred_element_type=jnp.float32) + b2[...]
    o_ref[...] = p_ref[...] + delta


def _coord(s, posp, w1, b1, w2, b2):
    fs = _full_spec
    return pl.pallas_call(
        _coord_body,
        grid=(_NB,),
        in_specs=[pl.BlockSpec((_BLK, _H), lambda i: (i, 0)),
                  pl.BlockSpec((_BLK, 3), lambda i: (i, 0)),
                  fs((_H, _H)), fs((1, _H)), fs((_H, 3)), fs((1, 3))],
        out_specs=pl.BlockSpec((_BLK, 3), lambda i: (i, 0)),
        out_shape=jax.ShapeDtypeStruct((_NPAD, 3), jnp.float32),
    )(s, posp, w1, b1, w2, b2)


def _painn_stack(s, layers, ea, d2, src2, dst2, bnd):
    for lp in layers:
        wm, bm = lp["msg"]
        wu, bu = lp["upd"]
        t = _mm(s, wm[:_H])
        c = _edge_const(ea, d2, wm[_H:_H + _ED], wm[_H + _ED:], bm[None, :])
        agg = _sc_edge(t, c, src2, dst2, bnd)
        s = _upd(s, agg, wu[:_H], wu[_H:], bu[None, :])
    return s


def kernel(z, vector_features, edge_index, edge_attr, pos, batch, params):
    del vector_features
    f32 = jnp.float32

    src = jnp.concatenate(
        [edge_index[0].astype(jnp.int32), jnp.zeros((_EPAD - _E,), jnp.int32)])
    dst = jnp.concatenate(
        [edge_index[1].astype(jnp.int32),
         jnp.full((_EPAD - _E,), _NPAD - 1, jnp.int32)])
    eap = jnp.concatenate(
        [edge_attr.astype(f32), jnp.zeros((_EPAD - _E, _ED), f32)])
    perm = jnp.argsort(dst)
    sdst = dst[perm]
    src2 = src[perm].reshape(_NCHUNK, _CHUNK)
    dst2 = sdst.reshape(_NCHUNK, _CHUNK)
    ea = eap[perm]
    bnd = jnp.searchsorted(sdst, jnp.arange(0, _NPAD + 1, _NPT)
                           ).astype(jnp.int32)
    bnd = jnp.concatenate([bnd, jnp.full((15,), _EPAD, jnp.int32)])

    posp = jnp.concatenate([pos.astype(f32), jnp.zeros((_NPAD - _N, 3), f32)])
    px, py, pz = posp[:, 0], posp[:, 1], posp[:, 2]

    z3 = jnp.concatenate(
        [z.astype(jnp.int32), jnp.full((_NPAD - _N,), _VOCAB, jnp.int32)]
    ).reshape(_NB, 1, _BLK)
    b_pad = jnp.concatenate(
        [batch.astype(jnp.int32), jnp.full((_NPAD - _N,), _B, jnp.int32)])
    b3 = b_pad.reshape(_NB, 1, _BLK)
    b3p = b_pad.reshape(_NPB, 1, _PBLK)

    emb = jnp.concatenate(
        [params["embed"].astype(f32), jnp.zeros((_VPAD - _VOCAB, _H), f32)])

    d2c = _sc_dist(px, py, pz, src2, dst2)
    d2 = d2c.reshape(_EPAD, 1)

    s = _embed(z3, emb)
    s = _painn_stack(s, params["enc_layers"], ea, d2, src2, dst2, bnd)

    sumf, cntf, maxf = _pool(b3p, s)
    w1, b1 = params["pool1"]
    w2, b2 = params["pool2"]
    g = _poolmlp(sumf, cntf, maxf, w1[:_H], w1[_H:2 * _H], w1[2 * _H:],
                 b1[None, :], w2, b2[None, :])

    wle1, ble1 = params["le1"]
    wle2, ble2 = params["le2"]
    wmu, bmu = params["mu"]
    wlv, blv = params["logvar"]
    mu, logvar = _latent(s, b3, g, wle1[:_H], wle1[_H:], ble1[None, :],
                         wle2, ble2[None, :], wmu, bmu[None, :],
                         wlv, blv[None, :])

    wd1, bd1 = params["ld1"]
    wd2, bd2 = params["ld2"]
    wd3, bd3 = params["ld3"]
    atom = _decmlp(mu, wd1, bd1[None, :], wd2, bd2[None, :], wd3, bd3[None, :])
    s2 = _painn_stack(atom, params["dec_layers"], ea, d2, src2, dst2, bnd)

    wc1, bc1 = params["coord1"]
    wc2, bc2 = params["coord2"]
    pos_pred = _coord(s2, posp, wc1, bc1[None, :], wc2, bc2[None, :])

    return (pos_pred[:_N], mu[:_N], logvar[:_N])


# speedup vs baseline: 4.0725x; 4.0725x over previous
import functools

import jax
import jax.numpy as jnp
import numpy as np
from jax import lax
from jax.experimental import pallas as pl
from jax.experimental.pallas import tpu as pltpu
from jax.experimental.pallas import tpu_sc as plsc

_N = 10000
_NPAD = 10240
_H = 128
_E = 320000
_ED = 19
_B = 64
_VOCAB = 101
_VPAD = 104
_LAT = 32

_NC, _NS = 2, 16
_NW = _NC * _NS
_CHUNK = 64
_CPW = 160
_SBC = 16
_NCHUNK = _NW * _CPW
_EPAD = _NCHUNK * _CHUNK
_RPT = _NPAD // _NS

_BLK = 1024
_NB = _NPAD // _BLK
_NEB = _EPAD // _BLK
_PBLK = 128
_NPB = _NPAD // _PBLK

_sc_mesh = plsc.VectorSubcoreMesh(core_axis_name="c", subcore_axis_name="s")
_sc_params = pltpu.CompilerParams(needs_layout_passes=False)

_PERM = np.stack(
    [np.arange(16)[None, :] + np.arange(0, _H, 32)[:, None],
     np.arange(16)[None, :] + 16 + np.arange(0, _H, 32)[:, None]],
    axis=-1).reshape(-1)


def _silu(x):
    return jax.nn.silu(x)


@functools.partial(
    pl.kernel,
    out_type=jax.ShapeDtypeStruct((_NCHUNK, _CHUNK), jnp.float32),
    mesh=_sc_mesh,
    compiler_params=_sc_params,
    scratch_types=[
        pltpu.VMEM((_NPAD,), jnp.float32),
        pltpu.VMEM((_NPAD,), jnp.float32),
        pltpu.VMEM((_NPAD,), jnp.float32),
        pltpu.VMEM((1, _CHUNK), jnp.int32),
        pltpu.VMEM((1, _CHUNK), jnp.int32),
        pltpu.VMEM((1, _CHUNK), jnp.float32),
    ],
)
def _sc_dist(px_hbm, py_hbm, pz_hbm, src_hbm, dst_hbm, out_hbm,
             px_v, py_v, pz_v, sb, db, d2b):
    wid = lax.axis_index("s") * _NC + lax.axis_index("c")
    pltpu.sync_copy(px_hbm, px_v)
    pltpu.sync_copy(py_hbm, py_v)
    pltpu.sync_copy(pz_hbm, pz_v)

    def chunk(i, carry):
        ci = wid * _CPW + i
        pltpu.sync_copy(src_hbm.at[ci], sb.at[0])
        pltpu.sync_copy(dst_hbm.at[ci], db.at[0])
        for j in range(_CHUNK // 16):
            sl = pl.ds(j * 16, 16)
            si = sb[0, sl]
            di = db[0, sl]
            dx = plsc.load_gather(px_v, [si]) - plsc.load_gather(px_v, [di])
            dy = plsc.load_gather(py_v, [si]) - plsc.load_gather(py_v, [di])
            dz = plsc.load_gather(pz_v, [si]) - plsc.load_gather(pz_v, [di])
            d2b[0, sl] = dx * dx + dy * dy + dz * dz
        pltpu.sync_copy(d2b.at[0], out_hbm.at[ci])
        return carry

    lax.fori_loop(0, _CPW, chunk, 0)


@functools.partial(
    pl.kernel,
    out_type=jax.ShapeDtypeStruct((_NC, _NPAD, _H), jnp.float32),
    mesh=_sc_mesh,
    compiler_params=_sc_params,
    scratch_types=[
        pltpu.VMEM_SHARED((_NPAD, _H), jnp.float32),
        pltpu.VMEM((_SBC, _CHUNK), jnp.int32),
        pltpu.VMEM((_SBC, _CHUNK), jnp.int32),
        pltpu.VMEM((_CHUNK, _H), jnp.float32),
        pltpu.VMEM((_CHUNK, _H), jnp.float32),
        pltpu.VMEM((_CHUNK, _H), jnp.float32),
        pltpu.VMEM((_CHUNK, _H), jnp.float32),
        pltpu.SemaphoreType.DMA,
        pltpu.SemaphoreType.DMA,
    ],
)
def _sc_edge(t_hbm, c_hbm, src_hbm, dst_hbm, out_hbm,
             acc, srcall, dstall, tv0, tv1, cv0, cv1, sem0, sem1):
    cid = lax.axis_index("c")
    sid = lax.axis_index("s")
    wid = sid * _NC + cid
    base = wid * _CPW

    def zrow(r, carry):
        for j in range(_H // 16):
            tv0[r, pl.ds(j * 16, 16)] = jnp.zeros((16,), jnp.float32)
        return carry

    lax.fori_loop(0, _CHUNK, zrow, 0)

    def zcp(k, carry):
        pltpu.sync_copy(tv0, acc.at[pl.ds(sid * _RPT + k * _CHUNK, _CHUNK)])
        return carry

    lax.fori_loop(0, _RPT // _CHUNK, zcp, 0)
    plsc.subcore_barrier()

    def fire(gi, li, tv, cv, sem):
        pltpu.async_copy(t_hbm.at[srcall.at[li]], tv, sem)
        pltpu.async_copy(c_hbm.at[pl.ds((base + gi) * _CHUNK, _CHUNK)], cv, sem)

    def drain(tv, cv, sem):
        pltpu.make_async_copy(t_hbm.at[pl.ds(0, _CHUNK)], tv, sem).wait()
        pltpu.make_async_copy(c_hbm.at[pl.ds(0, _CHUNK)], cv, sem).wait()

    def process(li, tv, cv):
        def row(r, c2):
            for j in range(_H // 16):
                sl = pl.ds(j * 16, 16)
                x = tv[r, sl] + cv[r, sl]
                tv[r, sl] = x / (1.0 + jnp.exp(-x))
            return c2

        lax.fori_loop(0, _CHUNK, row, 0)
        pltpu.sync_copy(tv, acc.at[dstall.at[li]], add=True)

    def superblock(sbi, carry):
        sb0 = sbi * _SBC
        pltpu.sync_copy(src_hbm.at[pl.ds(base + sb0, _SBC)], srcall)
        pltpu.sync_copy(dst_hbm.at[pl.ds(base + sb0, _SBC)], dstall)
        fire(sb0, 0, tv0, cv0, sem0)

        def pair(j, c2):
            i0 = 2 * j
            fire(sb0 + i0 + 1, i0 + 1, tv1, cv1, sem1)
            drain(tv0, cv0, sem0)
            process(i0, tv0, cv0)

            @pl.when(j < _SBC // 2 - 1)
            def _():
                fire(sb0 + i0 + 2, i0 + 2, tv0, cv0, sem0)

            drain(tv1, cv1, sem1)
            process(i0 + 1, tv1, cv1)
            return c2

        lax.fori_loop(0, _SBC // 2, pair, 0)
        return carry

    lax.fori_loop(0, _CPW // _SBC, superblock, 0)
    plsc.subcore_barrier()

    def ecp(k, carry):
        off = sid * _RPT + k * _CHUNK
        pltpu.sync_copy(acc.at[pl.ds(off, _CHUNK)], tv0)
        pltpu.sync_copy(tv0, out_hbm.at[cid, pl.ds(off, _CHUNK)])
        return carry

    lax.fori_loop(0, _RPT // _CHUNK, ecp, 0)


def _full_spec(shape):
    n = len(shape)
    return pl.BlockSpec(shape, lambda *_, _n=n: (0,) * _n)


def _embed_body(z_ref, emb_ref, o_ref):
    zb = z_ref[0, 0, :]
    oh = (zb[:, None] == lax.broadcasted_iota(jnp.int32, (_BLK, _VPAD), 1)
          ).astype(jnp.float32)
    o_ref[...] = jnp.dot(oh, emb_ref[...], preferred_element_type=jnp.float32)


def _embed(z3, emb):
    return pl.pallas_call(
        _embed_body,
        grid=(_NB,),
        in_specs=[pl.BlockSpec((1, 1, _BLK), lambda i: (i, 0, 0)),
                  _full_spec((_VPAD, _H))],
        out_specs=pl.BlockSpec((_BLK, _H), lambda i: (i, 0)),
        out_shape=jax.ShapeDtypeStruct((_NPAD, _H), jnp.float32),
    )(z3, emb)


def _mm_body(x_ref, w_ref, o_ref):
    o_ref[...] = jnp.dot(x_ref[...], w_ref[...],
                         preferred_element_type=jnp.float32)


def _mm(x, w):
    k, m = w.shape
    return pl.pallas_call(
        _mm_body,
        grid=(_NB,),
        in_specs=[pl.BlockSpec((_BLK, k), lambda i: (i, 0)),
                  _full_spec((k, m))],
        out_specs=pl.BlockSpec((_BLK, m), lambda i: (i, 0)),
        out_shape=jax.ShapeDtypeStruct((_NPAD, m), jnp.float32),
    )(x, w)


def _c_body(ea_ref, d2_ref, wea_ref, wd_ref, b_ref, o_ref):
    dist = jnp.sqrt(d2_ref[...])
    o_ref[...] = (jnp.dot(ea_ref[...], wea_ref[...],
                          preferred_element_type=jnp.float32)
                  + dist * wd_ref[...] + b_ref[...])


def _edge_const(ea, d2, wea, wd, b):
    return pl.pallas_call(
        _c_body,
        grid=(_NEB,),
        in_specs=[pl.BlockSpec((_BLK, _ED), lambda i: (i, 0)),
                  pl.BlockSpec((_BLK, 1), lambda i: (i, 0)),
                  _full_spec((_ED, _H)),
                  _full_spec((1, _H)),
                  _full_spec((1, _H))],
        out_specs=pl.BlockSpec((_BLK, _H), lambda i: (i, 0)),
        out_shape=jax.ShapeDtypeStruct((_EPAD, _H), jnp.float32),
    )(ea, d2, wea, wd, b)


def _upd_body(s_ref, a0_ref, a1_ref, w1_ref, w2_ref, b_ref, o_ref):
    s = s_ref[...]
    agg = a0_ref[0] + a1_ref[0]
    x = (jnp.dot(s, w1_ref[...], preferred_element_type=jnp.float32)
         + jnp.dot(agg, w2_ref[...], preferred_element_type=jnp.float32)
         + b_ref[...])
    o_ref[...] = s + _silu(x)


def _upd(s, agg2, w1, w2, b):
    return pl.pallas_call(
        _upd_body,
        grid=(_NB,),
        in_specs=[pl.BlockSpec((_BLK, _H), lambda i: (i, 0)),
                  pl.BlockSpec((1, _BLK, _H), lambda i: (0, i, 0)),
                  pl.BlockSpec((1, _BLK, _H), lambda i: (1, i, 0)),
                  _full_spec((_H, _H)),
                  _full_spec((_H, _H)),
                  _full_spec((1, _H))],
        out_specs=pl.BlockSpec((_BLK, _H), lambda i: (i, 0)),
        out_shape=jax.ShapeDtypeStruct((_NPAD, _H), jnp.float32),
    )(s, agg2, agg2, w1, w2, b)


def _pool_body(b_ref, s_ref, sum_ref, cnt_ref, max_ref):
    i = pl.program_id(0)

    @pl.when(i == 0)
    def _init():
        sum_ref[...] = jnp.zeros_like(sum_ref)
        cnt_ref[...] = jnp.zeros_like(cnt_ref)
        max_ref[...] = jnp.full_like(max_ref, -jnp.inf)

    bb = b_ref[0, 0, :]
    s = s_ref[...]
    ohb = bb[:, None] == lax.broadcasted_iota(jnp.int32, (_PBLK, _B), 1)
    oh = ohb.astype(jnp.float32)
    sum_ref[...] += lax.dot_general(oh, s, (((0,), (0,)), ((), ())),
                                    preferred_element_type=jnp.float32)
    cnt_ref[...] += jnp.sum(oh, axis=0)[:, None]
    pen = jnp.where(ohb, 0.0, -jnp.inf)
    cand = jnp.max(pen[:, :, None] + s[:, None, :], axis=0)
    max_ref[...] = jnp.maximum(max_ref[...], cand)


def _pool(b3, s):
    out3 = jax.ShapeDtypeStruct((_B, _H), jnp.float32)
    return pl.pallas_call(
        _pool_body,
        grid=(_NPB,),
        in_specs=[pl.BlockSpec((1, 1, _PBLK), lambda i: (i, 0, 0)),
                  pl.BlockSpec((_PBLK, _H), lambda i: (i, 0))],
        out_specs=(pl.BlockSpec((_B, _H), lambda i: (0, 0)),) * 3,
        out_shape=(out3, out3, out3),
    )(b3, s)


def _poolmlp_body(sum_ref, cnt_ref, max_ref, w1a, w1b, w1c, b1, w2, b2, g_ref):
    cnt = cnt_ref[...]
    mean = sum_ref[...] / jnp.maximum(cnt, 1.0)
    mx = jnp.where(cnt > 0, max_ref[...], 0.0)
    h = (jnp.dot(mean, w1a[...], preferred_element_type=jnp.float32)
         + jnp.dot(sum_ref[...], w1b[...], preferred_element_type=jnp.float32)
         + jnp.dot(mx, w1c[...], preferred_element_type=jnp.float32)
         + b1[...])
    h = _silu(h)
    g_ref[...] = jnp.dot(h, w2[...], preferred_element_type=jnp.float32) + b2[...]


def _poolmlp(sumf, cntf, maxf, w1a, w1b, w1c, b1, w2, b2):
    fs = _full_spec
    return pl.pallas_call(
        _poolmlp_body,
        in_specs=[fs((_B, _H))] * 3 + [fs((_H, _H))] * 3 + [fs((1, _H)),
                  fs((_H, _H)), fs((1, _H))],
        out_specs=fs((_B, _H)),
        out_shape=jax.ShapeDtypeStruct((_B, _H), jnp.float32),
    )(sumf, cntf, maxf, w1a, w1b, w1c, b1, w2, b2)


def _lat_body(s_ref, b_ref, g_ref, wa, wb, b1, w2, b2, wmu, bmu, wlv, blv,
              mu_ref, lv_ref):
    s = s_ref[...]
    bb = b_ref[0, 0, :]
    oh = (bb[:, None] == lax.broadcasted_iota(jnp.int32, (_BLK, _B), 1)
          ).astype(jnp.float32)
    gb = jnp.dot(oh, g_ref[...], preferred_element_type=jnp.float32)
    h = _silu(jnp.dot(s, wa[...], preferred_element_type=jnp.float32)
              + jnp.dot(gb, wb[...], preferred_element_type=jnp.float32)
              + b1[...])
    h = _silu(jnp.dot(h, w2[...], preferred_element_type=jnp.float32) + b2[...])
    mu_ref[...] = jnp.dot(h, wmu[...], preferred_element_type=jnp.float32) + bmu[...]
    lv_ref[...] = jnp.dot(h, wlv[...], preferred_element_type=jnp.float32) + blv[...]


def _latent(s, b3, g, wa, wb, b1, w2, b2, wmu, bmu, wlv, blv):
    fs = _full_spec
    out = jax.ShapeDtypeStruct((_NPAD, _LAT), jnp.float32)
    return pl.pallas_call(
        _lat_body,
        grid=(_NB,),
        in_specs=[pl.BlockSpec((_BLK, _H), lambda i: (i, 0)),
                  pl.BlockSpec((1, 1, _BLK), lambda i: (i, 0, 0)),
                  fs((_B, _H)), fs((_H, _H)), fs((_H, _H)), fs((1, _H)),
                  fs((_H, _H)), fs((1, _H)),
                  fs((_H, _LAT)), fs((1, _LAT)), fs((_H, _LAT)), fs((1, _LAT))],
        out_specs=(pl.BlockSpec((_BLK, _LAT), lambda i: (i, 0)),) * 2,
        out_shape=(out, out),
    )(s, b3, g, wa, wb, b1, w2, b2, wmu, bmu, wlv, blv)


def _dec_body(mu_ref, w1, b1, w2, b2, w3, b3, o_ref):
    a = _silu(jnp.dot(mu_ref[...], w1[...], preferred_element_type=jnp.float32)
              + b1[...])
    a = _silu(jnp.dot(a, w2[...], preferred_element_type=jnp.float32) + b2[...])
    o_ref[...] = jnp.dot(a, w3[...], preferred_element_type=jnp.float32) + b3[...]


def _decmlp(mu, w1, b1, w2, b2, w3, b3):
    fs = _full_spec
    return pl.pallas_call(
        _dec_body,
        grid=(_NB,),
        in_specs=[pl.BlockSpec((_BLK, _LAT), lambda i: (i, 0)),
                  fs((_LAT, _H)), fs((1, _H)), fs((_H, _H)), fs((1, _H)),
                  fs((_H, _H)), fs((1, _H))],
        out_specs=pl.BlockSpec((_BLK, _H), lambda i: (i, 0)),
        out_shape=jax.ShapeDtypeStruct((_NPAD, _H), jnp.float32),
    )(mu, w1, b1, w2, b2, w3, b3)


def _coord_body(s_ref, p_ref, w1, b1, w2, b2, o_ref):
    h = _silu(jnp.dot(s_ref[...], w1[...], preferred_element_type=jnp.float32)
              + b1[...])
    delta = jnp.dot(h, w2[...], preferred_element_type=jnp.float32) + b2[...]
    o_ref[...] = p_ref[...] + delta


def _coord(s, posp, w1, b1, w2, b2):
    fs = _full_spec
    return pl.pallas_call(
        _coord_body,
        grid=(_NB,),
        in_specs=[pl.BlockSpec((_BLK, _H), lambda i: (i, 0)),
                  pl.BlockSpec((_BLK, 3), lambda i: (i, 0)),
                  fs((_H, _H)), fs((1, _H)), fs((_H, 3)), fs((1, 3))],
        out_specs=pl.BlockSpec((_BLK, 3), lambda i: (i, 0)),
        out_shape=jax.ShapeDtypeStruct((_NPAD, 3), jnp.float32),
    )(s, posp, w1, b1, w2, b2)


def _painn_stack(s, layers, ea, d2, src2, dst2):
    for lp in layers:
        wm, bm = lp["msg"]
        wu, bu = lp["upd"]
        t = _mm(s, wm[:_H])
        c = _edge_const(ea, d2, wm[_H:_H + _ED], wm[_H + _ED:], bm[None, :])
        agg2 = _sc_edge(t, c, src2, dst2)
        s = _upd(s, agg2, wu[:_H], wu[_H:], bu[None, :])
    return s


def kernel(z, vector_features, edge_index, edge_attr, pos, batch, params):
    del vector_features
    f32 = jnp.float32

    src2 = jnp.concatenate(
        [edge_index[0].astype(jnp.int32),
         jnp.zeros((_EPAD - _E,), jnp.int32)]).reshape(_NCHUNK, _CHUNK)
    dst2 = jnp.concatenate(
        [edge_index[1].astype(jnp.int32),
         jnp.full((_EPAD - _E,), _NPAD - 1, jnp.int32)]
    ).reshape(_NCHUNK, _CHUNK)
    ea = jnp.concatenate(
        [edge_attr.astype(f32), jnp.zeros((_EPAD - _E, _ED), f32)])

    posp = jnp.concatenate([pos.astype(f32), jnp.zeros((_NPAD - _N, 3), f32)])
    px, py, pz = posp[:, 0], posp[:, 1], posp[:, 2]

    z3 = jnp.concatenate(
        [z.astype(jnp.int32), jnp.full((_NPAD - _N,), _VOCAB, jnp.int32)]
    ).reshape(_NB, 1, _BLK)
    b_pad = jnp.concatenate(
        [batch.astype(jnp.int32), jnp.full((_NPAD - _N,), _B, jnp.int32)])
    b3 = b_pad.reshape(_NB, 1, _BLK)
    b3p = b_pad.reshape(_NPB, 1, _PBLK)

    emb = jnp.concatenate(
        [params["embed"].astype(f32), jnp.zeros((_VPAD - _VOCAB, _H), f32)])

    d2c = _sc_dist(px, py, pz, src2, dst2)
    d2 = d2c.reshape(_EPAD, 1)

    s = _embed(z3, emb)
    s = _painn_stack(s, params["enc_layers"], ea, d2, src2, dst2)

    sumf, cntf, maxf = _pool(b3p, s)
    w1, b1 = params["pool1"]
    w2, b2 = params["pool2"]
    g = _poolmlp(sumf, cntf, maxf, w1[:_H], w1[_H:2 * _H], w1[2 * _H:],
                 b1[None, :], w2, b2[None, :])

    wle1, ble1 = params["le1"]
    wle2, ble2 = params["le2"]
    wmu, bmu = params["mu"]
    wlv, blv = params["logvar"]
    mu, logvar = _latent(s, b3, g, wle1[:_H], wle1[_H:], ble1[None, :],
                         wle2, ble2[None, :], wmu, bmu[None, :],
                         wlv, blv[None, :])

    wd1, bd1 = params["ld1"]
    wd2, bd2 = params["ld2"]
    wd3, bd3 = params["ld3"]
    atom = _decmlp(mu, wd1, bd1[None, :], wd2, bd2[None, :], wd3, bd3[None, :])
    s2 = _painn_stack(atom, params["dec_layers"], ea, d2, src2, dst2)

    wc1, bc1 = params["coord1"]
    wc2, bc2 = params["coord2"]
    pos_pred = _coord(s2, posp, wc1, bc1[None, :], wc2, bc2[None, :])

    return (pos_pred[:_N], mu[:_N], logvar[:_N])
